# baseline (speedup 1.0000x reference)
import functools

import jax
import jax.numpy as jnp
from jax import lax
from jax.experimental import pallas as pl
from jax.experimental.pallas import tpu as pltpu
from jax.experimental.pallas import tpu_sc as plsc

VOCAB = 1682
EMBED_DIM = 32
BATCH = 16384

_info = plsc.get_sparse_core_info()
_NC = 1
_NS, _L = _info.num_subcores, _info.num_lanes
_NW = _NC * _NS
_B_PER_W = BATCH // _NW
_GCHUNK = 128
_NG = _B_PER_W // _GCHUNK
_CT = _B_PER_W // 128
_RT = EMBED_DIM // 8


def _make_sc_gather():
    mesh = plsc.VectorSubcoreMesh(
        core_axis_name="c", subcore_axis_name="s", num_cores=_NC
    )

    @functools.partial(
        pl.kernel,
        mesh=mesh,
        out_type=jax.ShapeDtypeStruct((_RT, BATCH // 128, 8, 128), jnp.float32),
        scratch_types=[
            pltpu.VMEM((_NG, _GCHUNK), jnp.int32),
            pltpu.VMEM((_B_PER_W, EMBED_DIM), jnp.float32),
            pltpu.VMEM((_RT, _CT, 8, 128), jnp.float32),
            [pltpu.SemaphoreType.DMA] * _NG,
            pltpu.SemaphoreType.DMA,
        ],
        compiler_params=pltpu.CompilerParams(
            use_tc_tiling_on_sc=False,
            needs_layout_passes=False,
            disable_bounds_checks=True,
            disable_semaphore_checks=True,
        ),
    )
    def sc_gather(ids_hbm, table_hbm, out_hbm, idx_v, rows_p, tile_v, gsem, osem):
        wid = lax.axis_index("s") * _NC + lax.axis_index("c")

        pltpu.sync_copy(ids_hbm.at[wid], idx_v)

        gathers = [
            pltpu.async_copy(
                table_hbm.at[idx_v.at[j]],
                rows_p.at[pl.ds(j * _GCHUNK, _GCHUNK)],
                gsem[j],
            )
            for j in range(_NG)
        ]

        lanes = lax.iota(jnp.int32, _L)
        for c in range(_CT):
            gathers[c].wait()
            for r in range(_RT):
                for j2 in range(8):
                    col = jnp.full((_L,), 8 * r + j2, jnp.int32)
                    vals = [
                        plsc.load_gather(rows_p, [c * 128 + g * _L + lanes, col])
                        for g in range(128 // _L)
                    ]
                    for g in range(128 // _L):
                        tile_v[r, c, j2, pl.ds(g * _L, _L)] = vals[g]

        writes = [
            pltpu.async_copy(
                tile_v.at[r],
                out_hbm.at[r, pl.ds(wid * _CT, _CT)],
                osem,
            )
            for r in range(_RT)
        ]
        for w in writes:
            w.wait()

    return sc_gather


_sc_gather = _make_sc_gather()


def kernel(movie_id, table):
    ids = movie_id.reshape(_NW, _NG, _GCHUNK)
    table_p = table[1:]
    raw = _sc_gather(ids, table_p)
    return jnp.transpose(raw, (1, 3, 0, 2)).reshape(BATCH, EMBED_DIM)


# speedup vs baseline: 1.4346x; 1.4346x over previous
import functools

import jax
import jax.numpy as jnp
from jax import lax
from jax.experimental import pallas as pl
from jax.experimental.pallas import tpu as pltpu
from jax.experimental.pallas import tpu_sc as plsc

VOCAB = 1682
EMBED_DIM = 32
BATCH = 16384

_info = plsc.get_sparse_core_info()
_NC, _NS, _L = 1, _info.num_subcores, _info.num_lanes
_NW = _NC * _NS
_B_PER_W = BATCH // _NW
_GCHUNK = 128
_NG = _B_PER_W // _GCHUNK


def _make_sc_gather():
    mesh = plsc.VectorSubcoreMesh(
        core_axis_name="c", subcore_axis_name="s", num_cores=_NC
    )

    @functools.partial(
        pl.kernel,
        mesh=mesh,
        out_type=jax.ShapeDtypeStruct((BATCH, EMBED_DIM), jnp.float32),
        scratch_types=[
            pltpu.VMEM((_NG, _GCHUNK), jnp.int32),
            pltpu.VMEM((_B_PER_W, EMBED_DIM), jnp.float32),
            [pltpu.SemaphoreType.DMA] * _NG,
            pltpu.SemaphoreType.DMA,
        ],
        compiler_params=pltpu.CompilerParams(
            use_tc_tiling_on_sc=False,
            disable_bounds_checks=True,
            disable_semaphore_checks=True,
        ),
    )
    def sc_gather(ids_hbm, table_hbm, out_hbm, idx_v, rows_v, gsem, osem):
        wid = lax.axis_index("s") * _NC + lax.axis_index("c")
        base = wid * _B_PER_W

        pltpu.sync_copy(ids_hbm.at[wid], idx_v)

        shifted = table_hbm.at[pl.ds(1, VOCAB)]

        gathers = [
            pltpu.async_copy(
                shifted.at[idx_v.at[j]],
                rows_v.at[pl.ds(j * _GCHUNK, _GCHUNK)],
                gsem[j],
            )
            for j in range(_NG)
        ]
        writes = []
        for j in range(_NG):
            gathers[j].wait()
            writes.append(
                pltpu.async_copy(
                    rows_v.at[pl.ds(j * _GCHUNK, _GCHUNK)],
                    out_hbm.at[pl.ds(base + j * _GCHUNK, _GCHUNK)],
                    osem,
                )
            )
        for c in writes:
            c.wait()

    return sc_gather


_sc_gather = _make_sc_gather()


def kernel(movie_id, table):
    ids = movie_id.reshape(_NW, _NG, _GCHUNK)
    return _sc_gather(ids, table)
